# baseline (speedup 1.0000x reference)
import functools

import jax
import jax.numpy as jnp
from jax.experimental import pallas as pl
from jax.experimental.pallas import tpu as pltpu


_RATIO = 0.3
_RC = 256
_CJ = 512


def _stats_kernel(text_ref, out_ref):
    text = text_ref[...]
    B, L = text.shape
    m = jnp.max(text, axis=1, keepdims=True)
    ii = jax.lax.broadcasted_iota(jnp.int32, (B, L), 1)
    amax = jnp.min(jnp.where(text == m, ii, L), axis=1, keepdims=True)
    nval = jnp.sum((text != 0).astype(jnp.int32), axis=1, keepdims=True)
    out_ref[0:B, 0:1] = amax
    out_ref[0:B, 1:2] = nval


def _prefix_sum_row(x):
    L = x.shape[1]
    nchunks = L // _CJ
    r = jax.lax.broadcasted_iota(jnp.int32, (_CJ, _CJ), 0)
    c = jax.lax.broadcasted_iota(jnp.int32, (_CJ, _CJ), 1)
    tri = (r <= c).astype(jnp.float32)
    outs = []
    carry = jnp.zeros((1, 1), jnp.float32)
    for ci in range(nchunks):
        xc = x[:, ci * _CJ:(ci + 1) * _CJ]
        p = jnp.dot(xc, tri, preferred_element_type=jnp.float32)
        outs.append(p + carry)
        carry = carry + jnp.sum(xc, keepdims=True)
    return jnp.concatenate(outs, axis=1)


def _main_kernel(s_ref, gfeat_ref, gfrow_ref, attnrow_ref, text_ref,
                 w0t_ref, b0_ref, g0_ref, be0_ref, w1t_ref, b1_ref,
                 lwt_ref, lb_ref, fused_ref, pooled_ref,
                 h1_ref, sums_ref, *, B, L, D, E, k, kp):
    b = pl.program_id(0)
    amax_b = s_ref[b, 0]

    row = attnrow_ref[0, 0, :].reshape(1, L)
    lane = jax.lax.broadcasted_iota(jnp.int32, (1, L), 1)
    row = jnp.where((lane == 0) | (lane == amax_b), -1.0, row)
    tmask = (text_ref[0, 0, :].reshape(1, L) != 0)
    row = row * tmask.astype(jnp.float32)
    row = row + 0.0

    ibits = jax.lax.bitcast_convert_type(row, jnp.int32)
    ubits = jax.lax.bitcast_convert_type(row, jnp.uint32)
    ukey = jnp.where(ibits < 0, ~ubits, ubits | jnp.uint32(0x80000000))

    def _bisect(i, t):
        bit = jnp.uint32(1) << (jnp.uint32(31) - i.astype(jnp.uint32))
        cand = t | bit
        cnt = jnp.sum((ukey >= cand).astype(jnp.int32))
        return jnp.where(cnt >= k, cand, t)

    tkey = jax.lax.fori_loop(0, 32, _bisect, jnp.uint32(0))
    gt = (ukey > tkey)
    eq = (ukey == tkey)
    c_gt = jnp.sum(gt.astype(jnp.int32))
    need = (k - c_gt).astype(jnp.float32)
    pref_eq = _prefix_sum_row(eq.astype(jnp.float32))
    keep = gt | (eq & (pref_eq <= need))
    keepf = keep.astype(jnp.float32)
    pos = _prefix_sum_row(keepf) - 1.0

    gf = gfrow_ref[0, 0, :].reshape(1, D)
    gl = jnp.dot(gf, lwt_ref[...], preferred_element_type=jnp.float32)
    gl = gl + lb_ref[...].reshape(1, E)
    for bb in range(B):
        @pl.when(b == bb)
        def _():
            fused_ref[bb:bb + 1, :] = gl

    feats = gfeat_ref[0]
    w0t = w0t_ref[...]
    b0 = b0_ref[...].reshape(1, D)
    ssum = jnp.zeros((1, D), jnp.float32)
    ssq = jnp.zeros((1, D), jnp.float32)
    for rc in range(kp // _RC):
        pvals = (rc * _RC + jax.lax.broadcasted_iota(
            jnp.int32, (_RC, 1), 0)).astype(jnp.float32)
        selc = ((pos == pvals) & keep).astype(jnp.float32)
        local = jnp.dot(selc, feats, preferred_element_type=jnp.float32)
        s2 = jnp.sum(local * local, axis=1, keepdims=True)
        local = local * (1.0 / (jnp.sqrt(s2) + 1e-8))
        h0 = jnp.dot(local, w0t, preferred_element_type=jnp.float32) + b0
        h1_ref[b, rc * _RC:(rc + 1) * _RC, :] = h0
        gmask = (rc * _RC + jax.lax.broadcasted_iota(
            jnp.int32, (_RC, 1), 0)) < k
        gm = gmask.astype(jnp.float32)
        ssum = ssum + jnp.sum(h0 * gm, axis=0, keepdims=True)
        ssq = ssq + jnp.sum(h0 * h0 * gm, axis=0, keepdims=True)

    @pl.when(b == 0)
    def _():
        sums_ref[0:1, :] = ssum
        sums_ref[1:2, :] = ssq

    @pl.when(b > 0)
    def _():
        sums_ref[0:1, :] = sums_ref[0:1, :] + ssum
        sums_ref[1:2, :] = sums_ref[1:2, :] + ssq

    @pl.when(b == B - 1)
    def _():
        n = float(B * k)
        mu = sums_ref[0:1, :] / n
        var = sums_ref[1:2, :] / n - mu * mu
        scale = g0_ref[...].reshape(1, D) * jax.lax.rsqrt(var + 1e-5)
        shift = be0_ref[...].reshape(1, D) - mu * scale
        w1t = w1t_ref[...]
        b1 = b1_ref[...].reshape(1, E)
        neg_inf = jnp.float32(-jnp.inf)
        for b2 in range(B):
            lens = jnp.minimum(s_ref[b2, 1] - 2, k)
            pooled = jnp.full((1, E), neg_inf, jnp.float32)
            for rc in range(kp // _RC):
                hc = h1_ref[b2, rc * _RC:(rc + 1) * _RC, :]
                a = jnp.maximum(hc * scale + shift, 0.0)
                h2 = jnp.dot(a, w1t, preferred_element_type=jnp.float32)
                rows = rc * _RC + jax.lax.broadcasted_iota(
                    jnp.int32, (_RC, 1), 0)
                h2 = jnp.where(rows < lens, h2, neg_inf)
                pooled = jnp.maximum(pooled, jnp.max(h2, axis=0,
                                                     keepdims=True))
            pooled = pooled + b1
            pooled_ref[b2:b2 + 1, :] = pooled
            fused_ref[b2:b2 + 1, :] = fused_ref[b2:b2 + 1, :] + pooled


def kernel(gfeatures, features, text, atten, linear_W, linear_b,
           mlp_l0_W, mlp_l0_b, bn0_gamma, bn0_beta, mlp_l1_W, mlp_l1_b):
    del features
    B, L, D = gfeatures.shape
    E = linear_W.shape[0]
    k = int((L - 2) * _RATIO)
    kp = ((k + _RC - 1) // _RC) * _RC

    stats = pl.pallas_call(
        _stats_kernel,
        out_shape=jax.ShapeDtypeStruct((8, 128), jnp.int32),
    )(text)

    text3 = text.reshape(B, 1, L)
    gfrows = gfeatures.reshape(B * L, 1, D)
    attnrows = atten.reshape(B * L, 1, L)
    w0t = mlp_l0_W.T
    w1t = mlp_l1_W.T
    lwt = linear_W.T

    grid_spec = pltpu.PrefetchScalarGridSpec(
        num_scalar_prefetch=1,
        grid=(B,),
        in_specs=[
            pl.BlockSpec((1, L, D), lambda b, s: (b, 0, 0)),
            pl.BlockSpec((1, 1, D), lambda b, s: (b * L + s[b, 0], 0, 0)),
            pl.BlockSpec((1, 1, L), lambda b, s: (b * L + s[b, 0], 0, 0)),
            pl.BlockSpec((1, 1, L), lambda b, s: (b, 0, 0)),
            pl.BlockSpec((D, D), lambda b, s: (0, 0)),
            pl.BlockSpec((D,), lambda b, s: (0,)),
            pl.BlockSpec((D,), lambda b, s: (0,)),
            pl.BlockSpec((D,), lambda b, s: (0,)),
            pl.BlockSpec((D, E), lambda b, s: (0, 0)),
            pl.BlockSpec((E,), lambda b, s: (0,)),
            pl.BlockSpec((D, E), lambda b, s: (0, 0)),
            pl.BlockSpec((E,), lambda b, s: (0,)),
        ],
        out_specs=[
            pl.BlockSpec((B, E), lambda b, s: (0, 0)),
            pl.BlockSpec((B, E), lambda b, s: (0, 0)),
        ],
        scratch_shapes=[
            pltpu.VMEM((B, kp, D), jnp.float32),
            pltpu.VMEM((8, D), jnp.float32),
        ],
    )

    body = functools.partial(_main_kernel, B=B, L=L, D=D, E=E, k=k, kp=kp)
    fused, pooled = pl.pallas_call(
        body,
        grid_spec=grid_spec,
        out_shape=[
            jax.ShapeDtypeStruct((B, E), jnp.float32),
            jax.ShapeDtypeStruct((B, E), jnp.float32),
        ],
        compiler_params=pltpu.CompilerParams(
            dimension_semantics=("arbitrary",)),
    )(stats, gfeatures, gfrows, attnrows, text3,
      w0t, mlp_l0_b, bn0_gamma, bn0_beta, w1t, mlp_l1_b, lwt, linear_b)

    return (fused, pooled)


# speedup vs baseline: 13.7625x; 13.7625x over previous
import functools

import jax
import jax.numpy as jnp
from jax.experimental import pallas as pl
from jax.experimental.pallas import tpu as pltpu


_RATIO = 0.3
_RC = 256
_CJ = 512


def _stats_kernel(text_ref, out_ref):
    text = text_ref[...]
    B, L = text.shape
    m = jnp.max(text, axis=1, keepdims=True)
    ii = jax.lax.broadcasted_iota(jnp.int32, (B, L), 1)
    amax = jnp.min(jnp.where(text == m, ii, L), axis=1, keepdims=True)
    nval = jnp.sum((text != 0).astype(jnp.int32), axis=1, keepdims=True)
    out_ref[0:B, 0:1] = amax
    out_ref[0:B, 1:2] = nval


def _prefix_sum_row(x):
    L = x.shape[1]
    nchunks = L // _CJ
    r = jax.lax.broadcasted_iota(jnp.int32, (_CJ, _CJ), 0)
    c = jax.lax.broadcasted_iota(jnp.int32, (_CJ, _CJ), 1)
    tri = (r <= c).astype(jnp.float32)
    outs = []
    carry = jnp.zeros((1, 1), jnp.float32)
    for ci in range(nchunks):
        xc = x[:, ci * _CJ:(ci + 1) * _CJ]
        p = jnp.dot(xc, tri, preferred_element_type=jnp.float32)
        outs.append(p + carry)
        carry = carry + jnp.sum(xc, keepdims=True)
    return jnp.concatenate(outs, axis=1)


def _main_kernel(s_ref, gfeat_ref, gfrow_ref, attnrow_ref, text_ref,
                 w0t_ref, b0_ref, g0_ref, be0_ref, w1t_ref, b1_ref,
                 lwt_ref, lb_ref, fused_ref, pooled_ref,
                 h1_ref, sums_ref, *, B, L, D, E, k, kp):
    b = pl.program_id(0)
    amax_b = s_ref[b, 0]
    sub = amax_b - (amax_b // 8) * 8
    rsel = (jax.lax.broadcasted_iota(jnp.int32, (8, 1), 0) == sub)
    rself = rsel.astype(jnp.float32)

    row = jnp.sum(attnrow_ref[0] * rself, axis=0, keepdims=True)
    lane = jax.lax.broadcasted_iota(jnp.int32, (1, L), 1)
    row = jnp.where((lane == 0) | (lane == amax_b), -1.0, row)
    tmask = (text_ref[0, 0, :].reshape(1, L) != 0)
    row = row * tmask.astype(jnp.float32)
    row = row + 0.0

    ibits = jax.lax.bitcast_convert_type(row, jnp.int32)
    ubits = jax.lax.bitcast_convert_type(row, jnp.uint32)
    ukey = jnp.where(ibits < 0, ~ubits, ubits | jnp.uint32(0x80000000))

    def _bisect(i, t):
        bit = jnp.uint32(1) << (jnp.uint32(31) - i.astype(jnp.uint32))
        cand = t | bit
        cnt = jnp.sum((ukey >= cand).astype(jnp.int32))
        return jnp.where(cnt >= k, cand, t)

    tkey = jax.lax.fori_loop(0, 32, _bisect, jnp.uint32(0))
    gt = (ukey > tkey)
    eq = (ukey == tkey)
    c_gt = jnp.sum(gt.astype(jnp.int32))
    need = (k - c_gt).astype(jnp.float32)
    pref_eq = _prefix_sum_row(eq.astype(jnp.float32))
    keep = gt | (eq & (pref_eq <= need))
    keepf = keep.astype(jnp.float32)
    pos = _prefix_sum_row(keepf) - 1.0

    gf = jnp.sum(gfrow_ref[0] * rself, axis=0, keepdims=True)
    gl = jnp.dot(gf, lwt_ref[...], preferred_element_type=jnp.float32)
    gl = gl + lb_ref[...].reshape(1, E)
    for bb in range(B):
        @pl.when(b == bb)
        def _():
            fused_ref[bb:bb + 1, :] = gl

    feats = gfeat_ref[0]
    w0t = w0t_ref[...]
    b0 = b0_ref[...].reshape(1, D)
    ssum = jnp.zeros((1, D), jnp.float32)
    ssq = jnp.zeros((1, D), jnp.float32)
    for rc in range(kp // _RC):
        pvals = (rc * _RC + jax.lax.broadcasted_iota(
            jnp.int32, (_RC, 1), 0)).astype(jnp.float32)
        selc = ((pos == pvals) & keep).astype(jnp.float32)
        local = jnp.dot(selc, feats, preferred_element_type=jnp.float32)
        s2 = jnp.sum(local * local, axis=1, keepdims=True)
        local = local * (1.0 / (jnp.sqrt(s2) + 1e-8))
        h0 = jnp.dot(local, w0t, preferred_element_type=jnp.float32) + b0
        h1_ref[b, rc * _RC:(rc + 1) * _RC, :] = h0
        gmask = (rc * _RC + jax.lax.broadcasted_iota(
            jnp.int32, (_RC, 1), 0)) < k
        gm = gmask.astype(jnp.float32)
        ssum = ssum + jnp.sum(h0 * gm, axis=0, keepdims=True)
        ssq = ssq + jnp.sum(h0 * h0 * gm, axis=0, keepdims=True)

    @pl.when(b == 0)
    def _():
        sums_ref[0:1, :] = ssum
        sums_ref[1:2, :] = ssq

    @pl.when(b > 0)
    def _():
        sums_ref[0:1, :] = sums_ref[0:1, :] + ssum
        sums_ref[1:2, :] = sums_ref[1:2, :] + ssq

    @pl.when(b == B - 1)
    def _():
        n = float(B * k)
        mu = sums_ref[0:1, :] / n
        var = sums_ref[1:2, :] / n - mu * mu
        scale = g0_ref[...].reshape(1, D) * jax.lax.rsqrt(var + 1e-5)
        shift = be0_ref[...].reshape(1, D) - mu * scale
        w1t = w1t_ref[...]
        b1 = b1_ref[...].reshape(1, E)
        neg_inf = jnp.float32(-jnp.inf)
        for b2 in range(B):
            lens = jnp.minimum(s_ref[b2, 1] - 2, k)
            pooled = jnp.full((1, E), neg_inf, jnp.float32)
            for rc in range(kp // _RC):
                hc = h1_ref[b2, rc * _RC:(rc + 1) * _RC, :]
                a = jnp.maximum(hc * scale + shift, 0.0)
                h2 = jnp.dot(a, w1t, preferred_element_type=jnp.float32)
                rows = rc * _RC + jax.lax.broadcasted_iota(
                    jnp.int32, (_RC, 1), 0)
                h2 = jnp.where(rows < lens, h2, neg_inf)
                pooled = jnp.maximum(pooled, jnp.max(h2, axis=0,
                                                     keepdims=True))
            pooled = pooled + b1
            pooled_ref[b2:b2 + 1, :] = pooled
            fused_ref[b2:b2 + 1, :] = fused_ref[b2:b2 + 1, :] + pooled


def kernel(gfeatures, features, text, atten, linear_W, linear_b,
           mlp_l0_W, mlp_l0_b, bn0_gamma, bn0_beta, mlp_l1_W, mlp_l1_b):
    del features
    B, L, D = gfeatures.shape
    E = linear_W.shape[0]
    k = int((L - 2) * _RATIO)
    kp = ((k + _RC - 1) // _RC) * _RC

    stats = pl.pallas_call(
        _stats_kernel,
        out_shape=jax.ShapeDtypeStruct((8, 128), jnp.int32),
    )(text)

    text3 = text.reshape(B, 1, L)
    w0t = mlp_l0_W.T
    w1t = mlp_l1_W.T
    lwt = linear_W.T

    grid_spec = pltpu.PrefetchScalarGridSpec(
        num_scalar_prefetch=1,
        grid=(B,),
        in_specs=[
            pl.BlockSpec((1, L, D), lambda b, s: (b, 0, 0)),
            pl.BlockSpec((1, 8, D), lambda b, s: (b, s[b, 0] // 8, 0)),
            pl.BlockSpec((1, 8, L), lambda b, s: (b, s[b, 0] // 8, 0)),
            pl.BlockSpec((1, 1, L), lambda b, s: (b, 0, 0)),
            pl.BlockSpec((D, D), lambda b, s: (0, 0)),
            pl.BlockSpec((D,), lambda b, s: (0,)),
            pl.BlockSpec((D,), lambda b, s: (0,)),
            pl.BlockSpec((D,), lambda b, s: (0,)),
            pl.BlockSpec((D, E), lambda b, s: (0, 0)),
            pl.BlockSpec((E,), lambda b, s: (0,)),
            pl.BlockSpec((D, E), lambda b, s: (0, 0)),
            pl.BlockSpec((E,), lambda b, s: (0,)),
        ],
        out_specs=[
            pl.BlockSpec((B, E), lambda b, s: (0, 0)),
            pl.BlockSpec((B, E), lambda b, s: (0, 0)),
        ],
        scratch_shapes=[
            pltpu.VMEM((B, kp, D), jnp.float32),
            pltpu.VMEM((8, D), jnp.float32),
        ],
    )

    body = functools.partial(_main_kernel, B=B, L=L, D=D, E=E, k=k, kp=kp)
    fused, pooled = pl.pallas_call(
        body,
        grid_spec=grid_spec,
        out_shape=[
            jax.ShapeDtypeStruct((B, E), jnp.float32),
            jax.ShapeDtypeStruct((B, E), jnp.float32),
        ],
        compiler_params=pltpu.CompilerParams(
            dimension_semantics=("arbitrary",)),
    )(stats, gfeatures, gfeatures, atten, text3,
      w0t, mlp_l0_b, bn0_gamma, bn0_beta, w1t, mlp_l1_b, lwt, linear_b)

    return (fused, pooled)
